# baseline (speedup 1.0000x reference)
import functools
import math

import jax
import jax.numpy as jnp
from jax.experimental import pallas as pl
from jax.experimental.pallas import tpu as pltpu


def _fused_kernel(x_ref, wq_ref, wk_ref, wv_ref, o_ref,
                  xb_ref, ta_ref, q_ref, k_ref, v_ref,
                  *, B, T, N, H, d_head, n_ct, scale):
    D = x_ref.shape[3]
    R = B * T * N
    S = B * N
    C = q_ref.shape[1]
    HG = C // d_head
    CT = wq_ref.shape[1]
    i = pl.program_id(1)

    @pl.when(i == 0)
    def _pool():
        x = x_ref[...]
        m = jnp.max(x, axis=1, keepdims=True)
        e = jnp.exp(x - m)
        ta = jnp.sum(x * e, axis=1) / jnp.sum(e, axis=1)
        ta_ref[...] = ta.reshape(S, D).astype(jnp.bfloat16)
        xb_ref[...] = x.reshape(R, D).astype(jnp.bfloat16)

    xb = xb_ref[...]
    tb = ta_ref[...]
    wq = wq_ref[...].astype(jnp.bfloat16)
    wk = wk_ref[...].astype(jnp.bfloat16)
    wv = wv_ref[...].astype(jnp.bfloat16)
    col = pl.multiple_of(i * CT, CT)
    q_ref[:, pl.ds(col, CT)] = (
        jnp.dot(xb, wq, preferred_element_type=jnp.float32) * scale
    ).astype(jnp.bfloat16)
    k_ref[:, pl.ds(col, CT)] = jnp.dot(
        tb, wk, preferred_element_type=jnp.float32).astype(jnp.bfloat16)
    v_ref[:, pl.ds(col, CT)] = jnp.dot(
        tb, wv, preferred_element_type=jnp.float32).astype(jnp.bfloat16)

    @pl.when(i == n_ct - 1)
    def _attn():
        q = q_ref[...]
        k = k_ref[...]
        v = v_ref[...]
        CC = B * HG * N
        rh = (jax.lax.broadcasted_iota(jnp.int32, (CC, C), 0) % (HG * N)) // N
        lh = jax.lax.broadcasted_iota(jnp.int32, (CC, C), 1) // d_head
        hmask = rh == lh
        kb = jnp.broadcast_to(
            k.reshape(B, 1, N, C), (B, HG, N, C)).reshape(CC, C)
        vb = jnp.broadcast_to(
            v.reshape(B, 1, N, C), (B, HG, N, C)).reshape(CC, C)
        kbig = jnp.where(hmask, kb, jnp.zeros((), jnp.bfloat16))
        vbig = jnp.where(hmask, vb.astype(jnp.float32), 0.0)

        s = jax.lax.dot_general(q, kbig, (((1,), (1,)), ((), ())),
                                preferred_element_type=jnp.float32)
        rb = jax.lax.broadcasted_iota(jnp.int32, (R, CC), 0) // (T * N)
        cb = jax.lax.broadcasted_iota(jnp.int32, (R, CC), 1) // (HG * N)
        p = jnp.where(rb == cb, jnp.maximum(s, 0.0), 0.0)

        o = jnp.dot(p, vbig, preferred_element_type=jnp.float32)
        o = o.reshape(B, T, N, C) + v.reshape(B, 1, N, C).astype(jnp.float32)
        o_ref[...] = o.astype(o_ref.dtype)


def kernel(x, boxes_in_flat, wq, wk, wv):
    del boxes_in_flat
    B, T, N, D = x.shape
    H = 8
    d_head = D // H
    R = B * T * N
    S = B * N
    scale = 1.0 / math.sqrt(d_head)

    NG = 2
    C = D // NG
    CT = 128
    n_ct = C // CT

    kern = functools.partial(
        _fused_kernel, B=B, T=T, N=N, H=H, d_head=d_head, n_ct=n_ct,
        scale=scale)
    return pl.pallas_call(
        kern,
        out_shape=jax.ShapeDtypeStruct((B, T, N, D), x.dtype),
        grid=(NG, n_ct),
        in_specs=[
            pl.BlockSpec((B, T, N, D), lambda g, i: (0, 0, 0, 0)),
            pl.BlockSpec((D, CT), lambda g, i: (0, g * (D // NG // 128) + i)),
            pl.BlockSpec((D, CT), lambda g, i: (0, g * (D // NG // 128) + i)),
            pl.BlockSpec((D, CT), lambda g, i: (0, g * (D // NG // 128) + i)),
        ],
        out_specs=pl.BlockSpec((B, T, N, C), lambda g, i: (0, 0, 0, g)),
        scratch_shapes=[
            pltpu.VMEM((R, D), jnp.bfloat16),
            pltpu.VMEM((S, D), jnp.bfloat16),
            pltpu.VMEM((R, C), jnp.bfloat16),
            pltpu.VMEM((S, C), jnp.bfloat16),
            pltpu.VMEM((S, C), jnp.bfloat16),
        ],
        compiler_params=pltpu.CompilerParams(
            dimension_semantics=("arbitrary", "arbitrary")),
    )(x, wq, wk, wv)


# speedup vs baseline: 1.0056x; 1.0056x over previous
import functools
import math

import jax
import jax.numpy as jnp
from jax.experimental import pallas as pl
from jax.experimental.pallas import tpu as pltpu


def _fused_kernel(x_ref, wq_ref, wk_ref, wv_ref, o_ref,
                  xb_ref, ta_ref, q_ref, k_ref, v_ref,
                  *, B, T, N, H, d_head, n_k, scale):
    D = x_ref.shape[3]
    R = B * T * N
    S = B * N
    KT = wq_ref.shape[0]
    i = pl.program_id(0)

    @pl.when(i == 0)
    def _pool():
        x = x_ref[...]
        m = jnp.max(x, axis=1, keepdims=True)
        e = jnp.exp(x - m)
        ta = jnp.sum(x * e, axis=1) / jnp.sum(e, axis=1)
        ta_ref[...] = ta.reshape(S, D).astype(jnp.bfloat16)
        xb_ref[...] = x.reshape(R, D).astype(jnp.bfloat16)

    row = pl.multiple_of(i * KT, KT)
    xs = xb_ref[:, pl.ds(row, KT)]
    ts = ta_ref[:, pl.ds(row, KT)]
    wq = wq_ref[...].astype(jnp.bfloat16)
    wk = wk_ref[...].astype(jnp.bfloat16)
    wv = wv_ref[...].astype(jnp.bfloat16)
    pq = jnp.dot(xs, wq, preferred_element_type=jnp.float32)
    pk = jnp.dot(ts, wk, preferred_element_type=jnp.float32)
    pv = jnp.dot(ts, wv, preferred_element_type=jnp.float32)

    @pl.when(i == 0)
    def _init():
        q_ref[...] = pq
        k_ref[...] = pk
        v_ref[...] = pv

    @pl.when(i > 0)
    def _acc():
        q_ref[...] += pq
        k_ref[...] += pk
        v_ref[...] += pv

    @pl.when(i == n_k - 1)
    def _attn():
        q = (q_ref[...] * scale).astype(jnp.bfloat16)
        k = k_ref[...]
        v = v_ref[...]
        C = B * H * N
        rh = (jax.lax.broadcasted_iota(jnp.int32, (C, D), 0) % (H * N)) // N
        lh = jax.lax.broadcasted_iota(jnp.int32, (C, D), 1) // d_head
        hmask = rh == lh
        kb = jnp.broadcast_to(
            k.reshape(B, 1, N, D), (B, H, N, D)).reshape(C, D)
        vb = jnp.broadcast_to(
            v.reshape(B, 1, N, D), (B, H, N, D)).reshape(C, D)
        zero = jnp.zeros((), jnp.bfloat16)
        kbig = jnp.where(hmask, kb.astype(jnp.bfloat16), zero)
        vbig = jnp.where(hmask, vb.astype(jnp.bfloat16), zero)

        s = jax.lax.dot_general(q, kbig, (((1,), (1,)), ((), ())),
                                preferred_element_type=jnp.float32)
        rb = jax.lax.broadcasted_iota(jnp.int32, (R, C), 0) // (T * N)
        cb = jax.lax.broadcasted_iota(jnp.int32, (R, C), 1) // (H * N)
        p = jnp.where(rb == cb, jnp.maximum(s, 0.0),
                      0.0).astype(jnp.bfloat16)

        o = jnp.dot(p, vbig, preferred_element_type=jnp.float32)
        o = o.reshape(B, T, N, D) + v.reshape(B, 1, N, D)
        o_ref[...] = o.astype(o_ref.dtype)


def kernel(x, boxes_in_flat, wq, wk, wv):
    del boxes_in_flat
    B, T, N, D = x.shape
    H = 8
    d_head = D // H
    R = B * T * N
    S = B * N
    scale = 1.0 / math.sqrt(d_head)

    KT = 128
    n_k = D // KT

    kern = functools.partial(
        _fused_kernel, B=B, T=T, N=N, H=H, d_head=d_head, n_k=n_k,
        scale=scale)
    return pl.pallas_call(
        kern,
        out_shape=jax.ShapeDtypeStruct((B, T, N, D), x.dtype),
        grid=(n_k,),
        in_specs=[
            pl.BlockSpec((B, T, N, D), lambda i: (0, 0, 0, 0)),
            pl.BlockSpec((KT, D), lambda i: (i, 0)),
            pl.BlockSpec((KT, D), lambda i: (i, 0)),
            pl.BlockSpec((KT, D), lambda i: (i, 0)),
        ],
        out_specs=pl.BlockSpec((B, T, N, D), lambda i: (0, 0, 0, 0)),
        scratch_shapes=[
            pltpu.VMEM((R, D), jnp.bfloat16),
            pltpu.VMEM((S, D), jnp.bfloat16),
            pltpu.VMEM((R, D), jnp.float32),
            pltpu.VMEM((S, D), jnp.float32),
            pltpu.VMEM((S, D), jnp.float32),
        ],
        compiler_params=pltpu.CompilerParams(
            dimension_semantics=("arbitrary",)),
    )(x, wq, wk, wv)
